# baseline (speedup 1.0000x reference)
import jax
import jax.numpy as jnp
from jax import lax
from jax.experimental import pallas as pl
from jax.experimental.pallas import tpu as pltpu
from jax.experimental.pallas import tpu_sc as plsc

NUM_ITEMS_K = 1000000
EMBED_DIM_K = 64
BATCH_K = 16384

_INFO = plsc.get_sparse_core_info()
_NC = _INFO.num_cores
_NS = _INFO.num_subcores
_NW = _NC * _NS
_CL = 512
_NFULL = NUM_ITEMS_K // _CL
_TAIL = NUM_ITEMS_K - _NFULL * _CL
_KMAX = 62
_CAP = 64
_PIECE = 2048


def _emit_item(buf, l, b, j, colbuf_v, out1d, sem_out, iota16):
    lbc = jnp.full((16,), l, jnp.int32)
    for t in range(4):
        vals = plsc.load_gather(buf, [t * 16 + iota16, lbc])
        colbuf_v[pl.ds(j * 64 + t * 16, 16)] = vals
    pltpu.async_copy(
        colbuf_v.at[pl.ds(j * 64, 64)],
        out1d.at[pl.ds(b * 64, 64)],
        sem_out,
    )


def _drain(n, colbuf_v, out1d, sem_out):
    def body(_, carry):
        pltpu.make_async_copy(
            colbuf_v.at[pl.ds(0, 64)], out1d.at[pl.ds(0, 64)], sem_out
        ).wait()
        return carry

    lax.fori_loop(0, n, body, 0)


def _extract_bucket(buf, k, cnt, bkb_v, bki_v, colbuf_v, out1d, sem_out):
    iota16 = lax.iota(jnp.int32, 16)
    e = jnp.minimum(cnt, _CAP)
    n_groups = (e + 15) >> 4

    def grp(g, carry):
        gs = g * 16
        rem = e - gs
        vec_i = bki_v[pl.ds(k * _CAP + gs, 16)]
        vec_b = bkb_v[pl.ds(k * _CAP + gs, 16)]
        l_vec = vec_i & (_CL - 1)
        for j in range(16):
            @pl.when(rem > j)
            def _():
                _emit_item(buf, l_vec[j], vec_b[j], j, colbuf_v, out1d,
                           sem_out, iota16)
        _drain(jnp.minimum(rem, 16), colbuf_v, out1d, sem_out)
        return carry

    lax.fori_loop(0, n_groups, grp, 0)


def _extract_overflow(buf, k, ovn, ovb_v, ovi_v, colbuf_v, out1d, sem_out):
    iota16 = lax.iota(jnp.int32, 16)
    n_groups = (ovn + 15) >> 4

    def grp(g, carry):
        gs = g * 16
        vec_i = ovi_v[pl.ds(gs, 16)]
        vec_b = ovb_v[pl.ds(gs, 16)]
        m = jnp.logical_and((gs + iota16) < ovn, (vec_i >> 14) == k)
        npop = plsc.all_reduce_population_count(m)[0]

        @pl.when(npop > 0)
        def _():
            m32 = m.astype(jnp.int32)
            l_vec = vec_i & (_CL - 1)
            for j in range(16):
                @pl.when(m32[j] != 0)
                def _():
                    _emit_item(buf, l_vec[j], vec_b[j], j, colbuf_v, out1d,
                               sem_out, iota16)
            _drain(npop, colbuf_v, out1d, sem_out)

        return carry

    lax.fori_loop(0, n_groups, grp, 0)


def _gather_body(idx_hbm, tableT_hbm, tail_hbm, out1d, idx_piece_v, bkb_v,
                 bki_v, ovb_v, ovi_v, cur_v, bufs, tailbuf_v, colbuf_v,
                 cnt_sm, sem_in, sem_out):
    wid = lax.axis_index("s") * _NC + lax.axis_index("c")
    iota16 = lax.iota(jnp.int32, 16)
    ones16 = jnp.full((16,), 1, jnp.int32)
    lane0 = iota16 == 0
    c63 = jnp.full((16,), 63, jnp.int32)

    def start(slot, c):
        return pltpu.async_copy(
            tableT_hbm.at[:, pl.ds(c * _CL, _CL)], bufs.at[slot], sem_in
        )

    start(0, wid)
    start(1, wid + _NW)

    for t in range(4):
        cur_v[pl.ds(t * 16, 16)] = jnp.zeros((16,), jnp.int32)

    def ov_append(idxval, bval):
        oc = plsc.load_gather(cur_v, [c63], mask=lane0)
        plsc.store_scatter(ovi_v, [oc], jnp.full((16,), idxval, jnp.int32),
                           mask=lane0)
        plsc.store_scatter(ovb_v, [oc], jnp.full((16,), bval, jnp.int32),
                           mask=lane0)
        plsc.addupdate_scatter(cur_v, [c63], ones16, mask=lane0)

    def place_piece(p, carry):
        pltpu.sync_copy(idx_hbm.at[pl.ds(p * _PIECE, _PIECE)], idx_piece_v)

        def place_group(g, carry2):
            vec = idx_piece_v[pl.ds(g * 16, 16)]
            m = ((vec >> 9) & (_NW - 1)) == wid
            kv = vec >> 14
            bbase = p * _PIECE + g * 16
            npop = plsc.all_reduce_population_count(m)[0]

            @pl.when(npop == 1)
            def _():
                pos = plsc.load_gather(cur_v, [kv], mask=m)
                okm = jnp.logical_and(m, pos < _CAP)
                slot = kv * _CAP + pos
                plsc.store_scatter(bki_v, [slot], vec, mask=okm)
                plsc.store_scatter(bkb_v, [slot], bbase + iota16, mask=okm)
                plsc.addupdate_scatter(cur_v, [kv], ones16, mask=m)
                ovm = jnp.logical_and(m, pos >= _CAP)
                ovn = plsc.all_reduce_population_count(ovm)[0]

                @pl.when(ovn > 0)
                def _():
                    ov32 = ovm.astype(jnp.int32)
                    for j in range(16):
                        @pl.when(ov32[j] != 0)
                        def _():
                            ov_append(vec[j], bbase + j)

            @pl.when(npop > 1)
            def _():
                m32 = m.astype(jnp.int32)
                for j in range(16):
                    @pl.when(m32[j] != 0)
                    def _():
                        kb = jnp.full((16,), kv[j], jnp.int32)
                        pos = plsc.load_gather(cur_v, [kb], mask=lane0)
                        p0 = pos[0]
                        plsc.addupdate_scatter(cur_v, [kb], ones16,
                                               mask=lane0)

                        @pl.when(p0 < _CAP)
                        def _():
                            slot = kb * _CAP + pos
                            plsc.store_scatter(
                                bki_v, [slot],
                                jnp.full((16,), vec[j], jnp.int32),
                                mask=lane0)
                            plsc.store_scatter(
                                bkb_v, [slot],
                                jnp.full((16,), bbase + j, jnp.int32),
                                mask=lane0)

                        @pl.when(p0 >= _CAP)
                        def _():
                            ov_append(vec[j], bbase + j)
            return carry2

        lax.fori_loop(0, _PIECE // 16, place_group, 0)
        return carry

    lax.fori_loop(0, BATCH_K // _PIECE, place_piece, 0)

    for t in range(4):
        endv = cur_v[pl.ds(t * 16, 16)]
        for j in range(16):
            cnt_sm[t * 16 + j] = endv[j]

    def wait_chunk(slot):
        pltpu.make_async_copy(
            tableT_hbm.at[:, pl.ds(0, _CL)], bufs.at[slot], sem_in
        ).wait()

    def handle(buf, k):
        _extract_bucket(buf, k, cnt_sm[k], bkb_v, bki_v, colbuf_v, out1d,
                        sem_out)

        @pl.when(cnt_sm[63] > 0)
        def _():
            _extract_overflow(buf, k, cnt_sm[63], ovb_v, ovi_v, colbuf_v,
                              out1d, sem_out)

    def pair(k2, carry):
        for phase in range(2):
            k = 2 * k2 + phase
            c = k * _NW + wid

            @pl.when(c < _NFULL)
            def _():
                wait_chunk(phase)
                handle(bufs.at[phase], k)

                @pl.when(c + 2 * _NW < _NFULL)
                def _():
                    start(phase, c + 2 * _NW)

        return carry

    lax.fori_loop(0, _KMAX // 2, pair, 0)

    @pl.when(wid == (_NFULL % _NW))
    def _():
        pltpu.sync_copy(tail_hbm, tailbuf_v)
        kt = (_NFULL - (_NFULL % _NW)) // _NW
        handle(tailbuf_v, kt)


def kernel(item_inputs, itemEmbedding_weight):
    idx = item_inputs.astype(jnp.int32)
    mesh = plsc.VectorSubcoreMesh(core_axis_name="c", subcore_axis_name="s")
    f = pl.kernel(
        _gather_body,
        out_type=jax.ShapeDtypeStruct((BATCH_K * EMBED_DIM_K,), jnp.float32),
        mesh=mesh,
        scratch_types=[
            pltpu.VMEM((_PIECE,), jnp.int32),
            pltpu.VMEM((_KMAX * _CAP,), jnp.int32),
            pltpu.VMEM((_KMAX * _CAP,), jnp.int32),
            pltpu.VMEM((BATCH_K + 16,), jnp.int32),
            pltpu.VMEM((BATCH_K + 16,), jnp.int32),
            pltpu.VMEM((64,), jnp.int32),
            pltpu.VMEM((2, EMBED_DIM_K, _CL), jnp.float32),
            pltpu.VMEM((EMBED_DIM_K, _TAIL), jnp.float32),
            pltpu.VMEM((16 * EMBED_DIM_K,), jnp.float32),
            pltpu.SMEM((64,), jnp.int32),
            pltpu.SemaphoreType.DMA,
            pltpu.SemaphoreType.DMA,
        ],
        compiler_params=pltpu.CompilerParams(needs_layout_passes=False),
    )
    tableT = itemEmbedding_weight.T
    tail = lax.slice(tableT, (0, _NFULL * _CL), (EMBED_DIM_K, NUM_ITEMS_K))
    out1d = f(idx, tableT, tail)
    return out1d.reshape(BATCH_K, EMBED_DIM_K)


# speedup vs baseline: 1.1104x; 1.1104x over previous
import jax
import jax.numpy as jnp
from jax import lax
from jax.experimental import pallas as pl
from jax.experimental.pallas import tpu as pltpu
from jax.experimental.pallas import tpu_sc as plsc

NUM_ITEMS_K = 1000000
EMBED_DIM_K = 64
BATCH_K = 16384

_INFO = plsc.get_sparse_core_info()
_NC = _INFO.num_cores
_NS = _INFO.num_subcores
_NW = _NC * _NS
_CL = 512
_NFULL = NUM_ITEMS_K // _CL
_TAIL = NUM_ITEMS_K - _NFULL * _CL
_KMAX = 62
_PIECE = 2048


def _extract_bucket(buf, s, e, myb_v, myidx_v, colbuf_v, out1d, sem_out):
    iota16 = lax.iota(jnp.int32, 16)
    n_groups = (e - s + 15) >> 4

    def grp(g, carry):
        gs = s + g * 16
        rem = e - gs
        vec_i = myidx_v[pl.ds(gs, 16)]
        vec_b = myb_v[pl.ds(gs, 16)]
        l_vec = vec_i & 511
        for j in range(16):
            @pl.when(rem > j)
            def _():
                l = l_vec[j]
                b = vec_b[j]
                lbc = jnp.full((16,), l, jnp.int32)
                for t in range(4):
                    vals = plsc.load_gather(buf, [t * 16 + iota16, lbc])
                    colbuf_v[pl.ds(j * 64 + t * 16, 16)] = vals
                pltpu.async_copy(
                    colbuf_v.at[pl.ds(j * 64, 64)],
                    out1d.at[pl.ds(b * 64, 64)],
                    sem_out,
                )

        def drain(_, carry2):
            pltpu.make_async_copy(
                colbuf_v.at[pl.ds(0, 64)],
                out1d.at[pl.ds(0, 64)],
                sem_out,
            ).wait()
            return carry2

        lax.fori_loop(0, jnp.minimum(rem, 16), drain, 0)
        return carry

    lax.fori_loop(0, n_groups, grp, 0)


def _gather_body(idx_hbm, tableT_hbm, tail_hbm, out1d, idx_piece_v, myb_v,
                 myidx_v, hist_v, cur_v, bufs, tailbuf_v, colbuf_v, off_sm,
                 cur_sm, sem_in, sem_out):
    wid = lax.axis_index("s") * _NC + lax.axis_index("c")
    iota16 = lax.iota(jnp.int32, 16)
    ones16 = jnp.full((16,), 1, jnp.int32)
    lane0 = iota16 == 0

    def start(slot, c):
        return pltpu.async_copy(
            tableT_hbm.at[:, pl.ds(c * _CL, _CL)], bufs.at[slot], sem_in
        )

    start(0, wid)
    start(1, wid + _NW)

    for t in range(4):
        hist_v[pl.ds(t * 16, 16)] = jnp.zeros((16,), jnp.int32)

    def hist_piece(p, carry):
        pltpu.sync_copy(idx_hbm.at[pl.ds(p * _PIECE, _PIECE)], idx_piece_v)

        def hist_group(g, carry2):
            for u in range(2):
                vec = idx_piece_v[pl.ds((2 * g + u) * 16, 16)]
                m = ((vec >> 9) & (_NW - 1)) == wid
                plsc.addupdate_scatter(hist_v, [vec >> 14], ones16, mask=m)
            return carry2

        lax.fori_loop(0, _PIECE // 32, hist_group, 0)
        return carry

    lax.fori_loop(0, BATCH_K // _PIECE, hist_piece, 0)

    run = 0
    for t in range(4):
        v = hist_v[pl.ds(t * 16, 16)]
        cs = plsc.cumsum(v)
        excl = cs - v
        cur_v[pl.ds(t * 16, 16)] = excl + run
        for j in range(16):
            off_sm[t * 16 + j] = excl[j] + run
        run = run + cs[15]

    def place_piece(p, carry):
        pltpu.sync_copy(idx_hbm.at[pl.ds(p * _PIECE, _PIECE)], idx_piece_v)

        def place_group(g2, carry2):
          for u in range(2):
            g = 2 * g2 + u
            vec = idx_piece_v[pl.ds(g * 16, 16)]
            m = ((vec >> 9) & (_NW - 1)) == wid
            kv = vec >> 14
            bbase = p * _PIECE + g * 16
            npop = plsc.all_reduce_population_count(m)[0]

            @pl.when(npop == 1)
            def _():
                pos = plsc.load_gather(cur_v, [kv], mask=m)
                plsc.store_scatter(myidx_v, [pos], vec, mask=m)
                plsc.store_scatter(myb_v, [pos], bbase + iota16, mask=m)
                plsc.addupdate_scatter(cur_v, [kv], ones16, mask=m)

            @pl.when(npop > 1)
            def _():
                m32 = m.astype(jnp.int32)
                for j in range(16):
                    @pl.when(m32[j] != 0)
                    def _():
                        kb = jnp.full((16,), kv[j], jnp.int32)
                        pos = plsc.load_gather(cur_v, [kb], mask=lane0)
                        plsc.store_scatter(
                            myidx_v, [pos],
                            jnp.full((16,), vec[j], jnp.int32), mask=lane0)
                        plsc.store_scatter(
                            myb_v, [pos],
                            jnp.full((16,), bbase + j, jnp.int32), mask=lane0)
                        plsc.addupdate_scatter(cur_v, [kb], ones16, mask=lane0)
          return carry2

        lax.fori_loop(0, _PIECE // 32, place_group, 0)
        return carry

    lax.fori_loop(0, BATCH_K // _PIECE, place_piece, 0)

    for t in range(4):
        endv = cur_v[pl.ds(t * 16, 16)]
        for j in range(16):
            cur_sm[t * 16 + j] = endv[j]

    def wait_chunk(slot):
        pltpu.make_async_copy(
            tableT_hbm.at[:, pl.ds(0, _CL)], bufs.at[slot], sem_in
        ).wait()

    def pair(k2, carry):
        for phase in range(2):
            k = 2 * k2 + phase
            c = k * _NW + wid

            @pl.when(c < _NFULL)
            def _():
                wait_chunk(phase)
                _extract_bucket(bufs.at[phase], off_sm[k], cur_sm[k],
                                myb_v, myidx_v, colbuf_v, out1d, sem_out)

                @pl.when(c + 2 * _NW < _NFULL)
                def _():
                    start(phase, c + 2 * _NW)

        return carry

    lax.fori_loop(0, _KMAX // 2, pair, 0)

    @pl.when(wid == (_NFULL % _NW))
    def _():
        pltpu.sync_copy(tail_hbm, tailbuf_v)
        kt = (_NFULL - (_NFULL % _NW)) // _NW
        _extract_bucket(tailbuf_v, off_sm[kt], cur_sm[kt],
                        myb_v, myidx_v, colbuf_v, out1d, sem_out)


def kernel(item_inputs, itemEmbedding_weight):
    idx = item_inputs.astype(jnp.int32)
    mesh = plsc.VectorSubcoreMesh(core_axis_name="c", subcore_axis_name="s")
    f = pl.kernel(
        _gather_body,
        out_type=jax.ShapeDtypeStruct((BATCH_K * EMBED_DIM_K,), jnp.float32),
        mesh=mesh,
        scratch_types=[
            pltpu.VMEM((_PIECE,), jnp.int32),
            pltpu.VMEM((BATCH_K + 16,), jnp.int32),
            pltpu.VMEM((BATCH_K + 16,), jnp.int32),
            pltpu.VMEM((64,), jnp.int32),
            pltpu.VMEM((64,), jnp.int32),
            pltpu.VMEM((2, EMBED_DIM_K, _CL), jnp.float32),
            pltpu.VMEM((EMBED_DIM_K, _TAIL), jnp.float32),
            pltpu.VMEM((16 * EMBED_DIM_K,), jnp.float32),
            pltpu.SMEM((64,), jnp.int32),
            pltpu.SMEM((64,), jnp.int32),
            pltpu.SemaphoreType.DMA,
            pltpu.SemaphoreType.DMA,
        ],
        compiler_params=pltpu.CompilerParams(needs_layout_passes=False),
    )
    tableT = itemEmbedding_weight.T
    tail = lax.slice(tableT, (0, _NFULL * _CL), (EMBED_DIM_K, NUM_ITEMS_K))
    out1d = f(idx, tableT, tail)
    return out1d.reshape(BATCH_K, EMBED_DIM_K)


# speedup vs baseline: 1.1694x; 1.0531x over previous
import jax
import jax.numpy as jnp
from jax import lax
from jax.experimental import pallas as pl
from jax.experimental.pallas import tpu as pltpu
from jax.experimental.pallas import tpu_sc as plsc

NUM_ITEMS_K = 1000000
EMBED_DIM_K = 64
BATCH_K = 16384

_INFO = plsc.get_sparse_core_info()
_NC = _INFO.num_cores
_NS = _INFO.num_subcores
_NW = _NC * _NS
_CL = 512
_NFULL = NUM_ITEMS_K // _CL
_TAIL = NUM_ITEMS_K - _NFULL * _CL
_KMAX = 62
_PIECE = 2048


def _extract_bucket(buf, s, e, myb_v, myidx_v, colbuf_v, out1d, sem_out):
    iota16 = lax.iota(jnp.int32, 16)
    n_groups = (e - s + 15) >> 4

    def grp(g, carry):
        gs = s + g * 16
        rem = e - gs
        vec_i = myidx_v[pl.ds(gs, 16)]
        vec_b = myb_v[pl.ds(gs, 16)]
        l_vec = vec_i & 511
        for j in range(16):
            @pl.when(rem > j)
            def _():
                l = l_vec[j]
                b = vec_b[j]
                lbc = jnp.full((16,), l, jnp.int32)
                for t in range(4):
                    vals = plsc.load_gather(buf, [t * 16 + iota16, lbc])
                    colbuf_v[pl.ds(j * 64 + t * 16, 16)] = vals
                pltpu.async_copy(
                    colbuf_v.at[pl.ds(j * 64, 64)],
                    out1d.at[pl.ds(b * 64, 64)],
                    sem_out,
                )

        def drain(_, carry2):
            pltpu.make_async_copy(
                colbuf_v.at[pl.ds(0, 64)],
                out1d.at[pl.ds(0, 64)],
                sem_out,
            ).wait()
            return carry2

        lax.fori_loop(0, jnp.minimum(rem, 16), drain, 0)
        return carry

    lax.fori_loop(0, n_groups, grp, 0)


def _gather_body(idx_hbm, tableT_hbm, tail_hbm, out1d, idx_piece_v, myb_v,
                 myidx_v, hist_v, cur_v, bufs, tailbuf_v, colbuf_v, off_sm,
                 cur_sm, sem_in, sem_out):
    wid = lax.axis_index("s") * _NC + lax.axis_index("c")
    iota16 = lax.iota(jnp.int32, 16)
    ones16 = jnp.full((16,), 1, jnp.int32)
    lane0 = iota16 == 0

    def start(slot, c):
        return pltpu.async_copy(
            tableT_hbm.at[:, pl.ds(c * _CL, _CL)], bufs.at[slot], sem_in
        )

    start(0, wid)
    start(1, wid + _NW)

    pltpu.sync_copy(idx_hbm, idx_piece_v)

    for t in range(4):
        hist_v[pl.ds(t * 16, 16)] = jnp.zeros((16,), jnp.int32)

    def hist_group(g, carry2):
        for u in range(2):
            vec = idx_piece_v[pl.ds((2 * g + u) * 16, 16)]
            m = ((vec >> 9) & (_NW - 1)) == wid
            plsc.addupdate_scatter(hist_v, [vec >> 14], ones16, mask=m)
        return carry2

    lax.fori_loop(0, BATCH_K // 32, hist_group, 0)

    run = 0
    for t in range(4):
        v = hist_v[pl.ds(t * 16, 16)]
        cs = plsc.cumsum(v)
        excl = cs - v
        cur_v[pl.ds(t * 16, 16)] = excl + run
        for j in range(16):
            off_sm[t * 16 + j] = excl[j] + run
        run = run + cs[15]

    def place_group(g2, carry2):
        for u in range(2):
            g = 2 * g2 + u
            vec = idx_piece_v[pl.ds(g * 16, 16)]
            m = ((vec >> 9) & (_NW - 1)) == wid
            kv = vec >> 14
            bbase = g * 16
            npop = plsc.all_reduce_population_count(m)[0]

            @pl.when(npop == 1)
            def _():
                pos = plsc.load_gather(cur_v, [kv], mask=m)
                plsc.store_scatter(myidx_v, [pos], vec, mask=m)
                plsc.store_scatter(myb_v, [pos], bbase + iota16, mask=m)
                plsc.addupdate_scatter(cur_v, [kv], ones16, mask=m)

            @pl.when(npop > 1)
            def _():
                m32 = m.astype(jnp.int32)
                for j in range(16):
                    @pl.when(m32[j] != 0)
                    def _():
                        kb = jnp.full((16,), kv[j], jnp.int32)
                        pos = plsc.load_gather(cur_v, [kb], mask=lane0)
                        plsc.store_scatter(
                            myidx_v, [pos],
                            jnp.full((16,), vec[j], jnp.int32), mask=lane0)
                        plsc.store_scatter(
                            myb_v, [pos],
                            jnp.full((16,), bbase + j, jnp.int32), mask=lane0)
                        plsc.addupdate_scatter(cur_v, [kb], ones16, mask=lane0)
        return carry2

    lax.fori_loop(0, BATCH_K // 32, place_group, 0)

    for t in range(4):
        endv = cur_v[pl.ds(t * 16, 16)]
        for j in range(16):
            cur_sm[t * 16 + j] = endv[j]

    def wait_chunk(slot):
        pltpu.make_async_copy(
            tableT_hbm.at[:, pl.ds(0, _CL)], bufs.at[slot], sem_in
        ).wait()

    def pair(k2, carry):
        for phase in range(2):
            k = 2 * k2 + phase
            c = k * _NW + wid

            @pl.when(c < _NFULL)
            def _():
                wait_chunk(phase)
                _extract_bucket(bufs.at[phase], off_sm[k], cur_sm[k],
                                myb_v, myidx_v, colbuf_v, out1d, sem_out)

                @pl.when(c + 2 * _NW < _NFULL)
                def _():
                    start(phase, c + 2 * _NW)

        return carry

    lax.fori_loop(0, _KMAX // 2, pair, 0)

    @pl.when(wid == (_NFULL % _NW))
    def _():
        pltpu.sync_copy(tail_hbm, tailbuf_v)
        kt = (_NFULL - (_NFULL % _NW)) // _NW
        _extract_bucket(tailbuf_v, off_sm[kt], cur_sm[kt],
                        myb_v, myidx_v, colbuf_v, out1d, sem_out)


def kernel(item_inputs, itemEmbedding_weight):
    idx = item_inputs.astype(jnp.int32)
    mesh = plsc.VectorSubcoreMesh(core_axis_name="c", subcore_axis_name="s")
    f = pl.kernel(
        _gather_body,
        out_type=jax.ShapeDtypeStruct((BATCH_K * EMBED_DIM_K,), jnp.float32),
        mesh=mesh,
        scratch_types=[
            pltpu.VMEM((BATCH_K,), jnp.int32),
            pltpu.VMEM((BATCH_K + 16,), jnp.int32),
            pltpu.VMEM((BATCH_K + 16,), jnp.int32),
            pltpu.VMEM((64,), jnp.int32),
            pltpu.VMEM((64,), jnp.int32),
            pltpu.VMEM((2, EMBED_DIM_K, _CL), jnp.float32),
            pltpu.VMEM((EMBED_DIM_K, _TAIL), jnp.float32),
            pltpu.VMEM((16 * EMBED_DIM_K,), jnp.float32),
            pltpu.SMEM((64,), jnp.int32),
            pltpu.SMEM((64,), jnp.int32),
            pltpu.SemaphoreType.DMA,
            pltpu.SemaphoreType.DMA,
        ],
        compiler_params=pltpu.CompilerParams(needs_layout_passes=False),
    )
    tableT = itemEmbedding_weight.T
    tail = lax.slice(tableT, (0, _NFULL * _CL), (EMBED_DIM_K, NUM_ITEMS_K))
    out1d = f(idx, tableT, tail)
    return out1d.reshape(BATCH_K, EMBED_DIM_K)
